# baseline (speedup 1.0000x reference)
import dataclasses
import functools

import jax
import jax.numpy as jnp
from jax import lax
from jax.experimental import pallas as pl
from jax.experimental.pallas import tpu as pltpu
from jax.experimental.pallas import tpu_sc as plsc

N = 10000
E = 320000
FIN = 128
H = 8
FOUT = 16

NC = 2
NS = 16
LANES = 16
HSC = H // NC

CH = 128
NBUF = 4
EDGES_PER_TILE = 19968
HPP = 2
NPH = HSC // HPP
ACC_ROWS = HPP * N
ACC_PAD = 20480
DEN_PAD = ACC_PAD // LANES
ROWS_PER_TILE = 1280
LAST_ROWS = ACC_ROWS - (NS - 1) * ROWS_PER_TILE


ROW_BLK = 400


def _proj_scores_body(x_ref, w_ref, asrc_ref, atgt_ref, ones_ref,
                      proj_ref, ssrc_ref, stgt_ref):
    p = jnp.dot(x_ref[...], w_ref[...], preferred_element_type=jnp.float32)
    proj_ref[...] = p
    ones = ones_ref[...]
    ssrc_ref[...] = jnp.dot(p * asrc_ref[...], ones,
                            preferred_element_type=jnp.float32)
    stgt_ref[...] = jnp.dot(p * atgt_ref[...], ones,
                            preferred_element_type=jnp.float32)


def _proj_scores(x, W, a_src, a_tgt, ones_bd):
    return pl.pallas_call(
        _proj_scores_body,
        grid=(N // ROW_BLK,),
        in_specs=[
            pl.BlockSpec((ROW_BLK, FIN), lambda i: (i, 0)),
            pl.BlockSpec((FIN, H * FOUT), lambda i: (0, 0)),
            pl.BlockSpec((ROW_BLK, H * FOUT), lambda i: (i, 0)),
            pl.BlockSpec((ROW_BLK, H * FOUT), lambda i: (i, 0)),
            pl.BlockSpec((H * FOUT, H), lambda i: (0, 0)),
        ],
        out_specs=[
            pl.BlockSpec((ROW_BLK, H * FOUT), lambda i: (i, 0)),
            pl.BlockSpec((ROW_BLK, H), lambda i: (i, 0)),
            pl.BlockSpec((ROW_BLK, H), lambda i: (i, 0)),
        ],
        out_shape=[
            jax.ShapeDtypeStruct((N, H * FOUT), jnp.float32),
            jax.ShapeDtypeStruct((N, H), jnp.float32),
            jax.ShapeDtypeStruct((N, H), jnp.float32),
        ],
    )(x, W, a_src, a_tgt, ones_bd)


def _edge_body(es_hbm, et_hbm, ssrc_hbm, stgt_hbm, feats_hbm, out_hbm,
               src_tab, tgt_tab, sidx, tidx, gidx, tloc, rows, wvec,
               den_tab, idxlin, obuf, dsum_all, acc_out, acc_den,
               isem, gsem, ssem, dsem):
    c = lax.axis_index("c")
    s = lax.axis_index("s")

    zero16 = jnp.zeros((LANES,), jnp.float32)
    iota16 = lax.iota(jnp.int32, LANES)
    row0 = s * ROWS_PER_TILE
    e0 = s * EDGES_PER_TILE
    edge_cnt = jnp.where(s == NS - 1, E - (NS - 1) * EDGES_PER_TILE,
                         EDGES_PER_TILE)

    for k in range(DEN_PAD // CH):
        for j in range(0, CH, LANES):
            idxlin[k, pl.ds(j, LANES)] = iota16 + (k * CH + j)

    @pl.loop(0, NPH)
    def _phase(p):
        grp = c * NPH + p

        pltpu.sync_copy(ssrc_hbm.at[grp], src_tab)
        pltpu.sync_copy(stgt_hbm.at[grp], tgt_tab)

        @pl.loop(0, DEN_PAD, unroll=16)
        def _zero_den(i):
            den_tab[i, :] = zero16

        @pl.loop(0, ROWS_PER_TILE, unroll=16)
        def _zero_buf(i):
            obuf[i, :] = zero16

        pltpu.sync_copy(obuf.at[pl.ds(0, DEN_PAD // NS)],
                        acc_den.at[pl.ds(s * (DEN_PAD // NS), DEN_PAD // NS)])
        pltpu.sync_copy(obuf, acc_out.at[pl.ds(row0, ROWS_PER_TILE)])

        plsc.subcore_barrier()

        for b in range(NBUF):
            pltpu.async_copy(es_hbm.at[pl.ds(e0 + b * CH, CH)],
                             sidx.at[b], isem.at[b])
            pltpu.async_copy(et_hbm.at[pl.ds(e0 + b * CH, CH)],
                             tidx.at[b], isem.at[b])

        @pl.loop(0, edge_cnt, step=NBUF * CH)
        def _iter(ce):
            gd = []
            for b in range(NBUF):
                @pl.when(ce > 0)
                def _drain_prev():
                    for h in range(HPP):
                        pltpu.make_async_copy(
                            rows.at[b, h], acc_out.at[tloc.at[b, h]],
                            ssem.at[b]).wait()

                off = e0 + ce + b * CH
                pltpu.make_async_copy(es_hbm.at[pl.ds(off, CH)],
                                      sidx.at[b], isem.at[b]).wait()
                pltpu.make_async_copy(et_hbm.at[pl.ds(off, CH)],
                                      tidx.at[b], isem.at[b]).wait()
                for h in range(HPP):
                    hvec = jnp.full((LANES,), h, jnp.int32)

                    @pl.loop(0, CH, step=LANES)
                    def _score(j):
                        sv = sidx[b, pl.ds(j, LANES)]
                        tv = tidx[b, pl.ds(j, LANES)]
                        ss = plsc.load_gather(src_tab, [hvec, sv])
                        ts = plsc.load_gather(tgt_tab, [hvec, tv])
                        z = ss + ts
                        ez = jnp.exp(z)
                        arg = jnp.where(z > 0, z, 0.2 * ez - 0.2)
                        w = jnp.exp(arg)
                        tl = tv + h * N
                        gidx[b, h, pl.ds(j, LANES)] = sv + (grp * HPP + h) * N
                        tloc[b, h, pl.ds(j, LANES)] = tl
                        wvec[b, h, pl.ds(j, LANES)] = w
                        plsc.addupdate_scatter(
                            den_tab, [tl >> 4, tl & (LANES - 1)], w)

                    gd.append(pltpu.async_copy(feats_hbm.at[gidx.at[b, h]],
                                               rows.at[b, h], gsem.at[b]))

                @pl.when(ce + NBUF * CH < edge_cnt)
                def _prefetch():
                    noff = off + NBUF * CH
                    pltpu.async_copy(es_hbm.at[pl.ds(noff, CH)],
                                     sidx.at[b], isem.at[b])
                    pltpu.async_copy(et_hbm.at[pl.ds(noff, CH)],
                                     tidx.at[b], isem.at[b])

            for b in range(NBUF):
                gd[HPP * b].wait()
                gd[HPP * b + 1].wait()
                for h in range(HPP):
                    @plsc.parallel_loop(0, CH, step=LANES, unroll=2)
                    def _scale(jj):
                        wv = wvec[b, h, pl.ds(jj, LANES)]
                        for i in range(LANES):
                            rows[b, h, jj + i, :] = (
                                rows[b, h, jj + i, :]
                                * jnp.broadcast_to(wv[i], (LANES,)))

                    pltpu.async_copy(
                        rows.at[b, h], acc_out.at[tloc.at[b, h]],
                        ssem.at[b], add=True)

        for b in range(NBUF):
            for h in range(HPP):
                pltpu.make_async_copy(
                    rows.at[b, h], acc_out.at[tloc.at[b, h]],
                    ssem.at[b]).wait()

        plsc.subcore_barrier()

        dd = []
        for k in range(DEN_PAD // CH):
            dd.append(pltpu.async_copy(
                den_tab.at[pl.ds(k * CH, CH)], acc_den.at[idxlin.at[k]],
                dsem, add=True))
        for d in dd:
            d.wait()

        plsc.subcore_barrier()

        out0 = grp * ACC_ROWS + row0
        pltpu.sync_copy(acc_den.at[pl.ds(s * (DEN_PAD // NS), DEN_PAD // NS)],
                        dsum_all)
        pltpu.sync_copy(acc_out.at[pl.ds(row0, ROWS_PER_TILE)], obuf)

        @pl.loop(0, ROWS_PER_TILE, step=LANES)
        def _div(jj):
            dv = dsum_all[jj >> 4, :]
            rv = 1.0 / jnp.where(dv == 0.0, 1.0, dv)
            for i in range(LANES):
                obuf[jj + i, :] = (obuf[jj + i, :]
                                   * jnp.broadcast_to(rv[i], (LANES,)))

        @pl.when(s < NS - 1)
        def _write_full():
            pltpu.sync_copy(obuf, out_hbm.at[pl.ds(out0, ROWS_PER_TILE)])

        @pl.when(s == NS - 1)
        def _write_last():
            pltpu.sync_copy(obuf.at[pl.ds(0, LAST_ROWS)],
                            out_hbm.at[pl.ds(out0, LAST_ROWS)])


def _edge_pass(es, et, scores_src, scores_tgt, feats_rows):
    mesh = plsc.VectorSubcoreMesh(core_axis_name="c", subcore_axis_name="s",
                                  num_cores=NC, num_subcores=NS)
    f32 = jnp.float32
    i32 = jnp.int32
    cp = pltpu.CompilerParams()
    fields = pltpu.CompilerParams.__dataclass_fields__
    if "needs_layout_passes" in fields:
        cp = dataclasses.replace(cp, needs_layout_passes=False)
    if "use_tc_tiling_on_sc" in fields:
        cp = dataclasses.replace(cp, use_tc_tiling_on_sc=False)
    kern = pl.kernel(
        _edge_body,
        out_type=jax.ShapeDtypeStruct((H * N, FOUT), f32),
        mesh=mesh,
        scratch_types=[
            pltpu.VMEM((HPP, N), f32),
            pltpu.VMEM((HPP, N), f32),
            pltpu.VMEM((NBUF, CH), i32),
            pltpu.VMEM((NBUF, CH), i32),
            pltpu.VMEM((NBUF, HPP, CH), i32),
            pltpu.VMEM((NBUF, HPP, CH), i32),
            pltpu.VMEM((NBUF, HPP, CH, FOUT), f32),
            pltpu.VMEM((NBUF, HPP, CH), f32),
            pltpu.VMEM((DEN_PAD, FOUT), f32),
            pltpu.VMEM((DEN_PAD // CH, CH), i32),
            pltpu.VMEM((ROWS_PER_TILE, FOUT), f32),
            pltpu.VMEM((DEN_PAD // NS, FOUT), f32),
            pltpu.VMEM_SHARED((ACC_PAD, FOUT), f32),
            pltpu.VMEM_SHARED((DEN_PAD, FOUT), f32),
            pltpu.SemaphoreType.DMA((NBUF,)),
            pltpu.SemaphoreType.DMA((NBUF,)),
            pltpu.SemaphoreType.DMA((NBUF,)),
            pltpu.SemaphoreType.DMA,
        ],
        compiler_params=cp,
    )
    return kern(es, et, scores_src, scores_tgt, feats_rows)


def kernel(x, edge_index, W, source_scorer, target_scorer):
    x = x.astype(jnp.float32)
    W = W.astype(jnp.float32)

    row_src = jnp.tile(source_scorer.reshape(H, FOUT), (1, H))
    row_tgt = jnp.tile(target_scorer.reshape(H, FOUT), (1, H))
    a_src = jnp.repeat(row_src, N // H, axis=0)
    a_tgt = jnp.repeat(row_tgt, N // H, axis=0)
    ones_bd = (jnp.arange(H * FOUT)[:, None] // FOUT ==
               jnp.arange(H)[None, :]).astype(jnp.float32)

    proj, ssrc, stgt = _proj_scores(x, W, a_src, a_tgt, ones_bd)

    feats_rows = proj.reshape(H * N, FOUT)
    scores_src = ssrc.reshape(H // HPP, HPP, N)
    scores_tgt = stgt.reshape(H // HPP, HPP, N)

    ei = edge_index.astype(jnp.int32)
    es = ei[:, 0]
    et = ei[:, 1]

    out = _edge_pass(es, et, scores_src, scores_tgt, feats_rows)
    return out.reshape(H, N, FOUT)


# speedup vs baseline: 1.1095x; 1.1095x over previous
import dataclasses
import functools

import jax
import jax.numpy as jnp
from jax import lax
from jax.experimental import pallas as pl
from jax.experimental.pallas import tpu as pltpu
from jax.experimental.pallas import tpu_sc as plsc

N = 10000
E = 320000
FIN = 128
H = 8
FOUT = 16

NC = 2
NS = 16
LANES = 16
HSC = H // NC

CH = 128
NBUF = 4
EDGES_PER_TILE = 19968
HPP = 2
NPH = HSC // HPP
ACC_ROWS = HPP * N
ACC_PAD = 20480
DEN_PAD = ACC_PAD // LANES
ROWS_PER_TILE = 1280
LAST_ROWS = ACC_ROWS - (NS - 1) * ROWS_PER_TILE


ROW_BLK = 400


def _proj_scores_body(x_ref, w_ref, asrc_ref, atgt_ref, ones_ref,
                      proj_ref, ssrc_ref, stgt_ref):
    p = jnp.dot(x_ref[...], w_ref[...], preferred_element_type=jnp.float32)
    proj_ref[...] = p
    ones = ones_ref[...]
    ssrc_ref[...] = jnp.dot(p * asrc_ref[...], ones,
                            preferred_element_type=jnp.float32)
    stgt_ref[...] = jnp.dot(p * atgt_ref[...], ones,
                            preferred_element_type=jnp.float32)


def _proj_scores(x, W, a_src, a_tgt, ones_bd):
    return pl.pallas_call(
        _proj_scores_body,
        grid=(N // ROW_BLK,),
        in_specs=[
            pl.BlockSpec((ROW_BLK, FIN), lambda i: (i, 0)),
            pl.BlockSpec((FIN, H * FOUT), lambda i: (0, 0)),
            pl.BlockSpec((ROW_BLK, H * FOUT), lambda i: (i, 0)),
            pl.BlockSpec((ROW_BLK, H * FOUT), lambda i: (i, 0)),
            pl.BlockSpec((H * FOUT, H), lambda i: (0, 0)),
        ],
        out_specs=[
            pl.BlockSpec((ROW_BLK, H * FOUT), lambda i: (i, 0)),
            pl.BlockSpec((ROW_BLK, H), lambda i: (i, 0)),
            pl.BlockSpec((ROW_BLK, H), lambda i: (i, 0)),
        ],
        out_shape=[
            jax.ShapeDtypeStruct((N, H * FOUT), jnp.float32),
            jax.ShapeDtypeStruct((N, H), jnp.float32),
            jax.ShapeDtypeStruct((N, H), jnp.float32),
        ],
    )(x, W, a_src, a_tgt, ones_bd)


def _edge_body(es_hbm, et_hbm, ssrc_hbm, stgt_hbm, feats_hbm, out_hbm,
               src_tab, tgt_tab, sidx, tidx, gidx, tloc, rows, wvec,
               den_tab, idxlin, obuf, dsum_all, acc_out, acc_den,
               isem, gsem, ssem, dsem):
    c = lax.axis_index("c")
    s = lax.axis_index("s")

    zero16 = jnp.zeros((LANES,), jnp.float32)
    iota16 = lax.iota(jnp.int32, LANES)
    row0 = s * ROWS_PER_TILE
    e0 = s * EDGES_PER_TILE
    edge_cnt = jnp.where(s == NS - 1, E - (NS - 1) * EDGES_PER_TILE,
                         EDGES_PER_TILE)

    for k in range(DEN_PAD // CH):
        for j in range(0, CH, LANES):
            idxlin[k, pl.ds(j, LANES)] = iota16 + (k * CH + j)

    @pl.loop(0, NPH)
    def _phase(p):
        grp = c * NPH + p

        pltpu.sync_copy(ssrc_hbm.at[grp], src_tab)
        pltpu.sync_copy(stgt_hbm.at[grp], tgt_tab)

        @pl.loop(0, DEN_PAD, unroll=16)
        def _zero_den(i):
            den_tab[i, :] = zero16

        @pl.loop(0, ROWS_PER_TILE, unroll=16)
        def _zero_buf(i):
            obuf[i, :] = zero16

        pltpu.sync_copy(obuf.at[pl.ds(0, DEN_PAD // NS)],
                        acc_den.at[pl.ds(s * (DEN_PAD // NS), DEN_PAD // NS)])
        pltpu.sync_copy(obuf, acc_out.at[pl.ds(row0, ROWS_PER_TILE)])

        plsc.subcore_barrier()

        for b in range(NBUF):
            pltpu.async_copy(es_hbm.at[pl.ds(e0 + b * CH, CH)],
                             sidx.at[b], isem.at[b])
            pltpu.async_copy(et_hbm.at[pl.ds(e0 + b * CH, CH)],
                             tidx.at[b], isem.at[b])

        @pl.loop(0, edge_cnt, step=NBUF * CH)
        def _iter(ce):
            gd = []
            for b in range(NBUF):
                @pl.when(ce > 0)
                def _drain_prev():
                    for h in range(HPP):
                        pltpu.make_async_copy(
                            rows.at[b, h], acc_out.at[tloc.at[b, h]],
                            ssem.at[b]).wait()

                off = e0 + ce + b * CH
                pltpu.make_async_copy(es_hbm.at[pl.ds(off, CH)],
                                      sidx.at[b], isem.at[b]).wait()
                pltpu.make_async_copy(et_hbm.at[pl.ds(off, CH)],
                                      tidx.at[b], isem.at[b]).wait()
                for h in range(HPP):
                    hvec = jnp.full((LANES,), h, jnp.int32)

                    @pl.loop(0, CH, step=LANES)
                    def _score(j):
                        sv = sidx[b, pl.ds(j, LANES)]
                        tv = tidx[b, pl.ds(j, LANES)]
                        ss = plsc.load_gather(src_tab, [hvec, sv])
                        ts = plsc.load_gather(tgt_tab, [hvec, tv])
                        z = ss + ts
                        ez = jnp.exp(z)
                        arg = jnp.where(z > 0, z, 0.2 * ez - 0.2)
                        w = jnp.exp(arg)
                        tl = tv + h * N
                        gidx[b, h, pl.ds(j, LANES)] = sv + (grp * HPP + h) * N
                        tloc[b, h, pl.ds(j, LANES)] = tl
                        wvec[b, h, pl.ds(j, LANES)] = w
                        plsc.addupdate_scatter(
                            den_tab, [tl >> 4, tl & (LANES - 1)], w)

                    gd.append(pltpu.async_copy(feats_hbm.at[gidx.at[b, h]],
                                               rows.at[b, h], gsem.at[b]))

                @pl.when(ce + NBUF * CH < edge_cnt)
                def _prefetch():
                    noff = off + NBUF * CH
                    pltpu.async_copy(es_hbm.at[pl.ds(noff, CH)],
                                     sidx.at[b], isem.at[b])
                    pltpu.async_copy(et_hbm.at[pl.ds(noff, CH)],
                                     tidx.at[b], isem.at[b])

            for b in range(NBUF):
                gd[HPP * b].wait()
                gd[HPP * b + 1].wait()
                for h in range(HPP):
                    @plsc.parallel_loop(0, CH, step=LANES)
                    def _scale(jj):
                        wv = wvec[b, h, pl.ds(jj, LANES)]
                        for i in range(LANES):
                            rows[b, h, jj + i, :] = (
                                rows[b, h, jj + i, :]
                                * jnp.broadcast_to(wv[i], (LANES,)))

                    pltpu.async_copy(
                        rows.at[b, h], acc_out.at[tloc.at[b, h]],
                        ssem.at[b], add=True)

        for b in range(NBUF):
            for h in range(HPP):
                pltpu.make_async_copy(
                    rows.at[b, h], acc_out.at[tloc.at[b, h]],
                    ssem.at[b]).wait()

        plsc.subcore_barrier()

        dd = []
        for k in range(DEN_PAD // CH):
            dd.append(pltpu.async_copy(
                den_tab.at[pl.ds(k * CH, CH)], acc_den.at[idxlin.at[k]],
                dsem, add=True))
        for d in dd:
            d.wait()

        plsc.subcore_barrier()

        out0 = grp * ACC_ROWS + row0
        pltpu.sync_copy(acc_den.at[pl.ds(s * (DEN_PAD // NS), DEN_PAD // NS)],
                        dsum_all)
        pltpu.sync_copy(acc_out.at[pl.ds(row0, ROWS_PER_TILE)], obuf)

        @pl.loop(0, ROWS_PER_TILE, step=LANES)
        def _div(jj):
            dv = dsum_all[jj >> 4, :]
            rv = 1.0 / jnp.where(dv == 0.0, 1.0, dv)
            for i in range(LANES):
                obuf[jj + i, :] = (obuf[jj + i, :]
                                   * jnp.broadcast_to(rv[i], (LANES,)))

        @pl.when(s < NS - 1)
        def _write_full():
            pltpu.sync_copy(obuf, out_hbm.at[pl.ds(out0, ROWS_PER_TILE)])

        @pl.when(s == NS - 1)
        def _write_last():
            pltpu.sync_copy(obuf.at[pl.ds(0, LAST_ROWS)],
                            out_hbm.at[pl.ds(out0, LAST_ROWS)])


def _edge_pass(es, et, scores_src, scores_tgt, feats_rows):
    mesh = plsc.VectorSubcoreMesh(core_axis_name="c", subcore_axis_name="s",
                                  num_cores=NC, num_subcores=NS)
    f32 = jnp.float32
    i32 = jnp.int32
    cp = pltpu.CompilerParams()
    fields = pltpu.CompilerParams.__dataclass_fields__
    if "needs_layout_passes" in fields:
        cp = dataclasses.replace(cp, needs_layout_passes=False)
    if "use_tc_tiling_on_sc" in fields:
        cp = dataclasses.replace(cp, use_tc_tiling_on_sc=False)
    kern = pl.kernel(
        _edge_body,
        out_type=jax.ShapeDtypeStruct((H * N, FOUT), f32),
        mesh=mesh,
        scratch_types=[
            pltpu.VMEM((HPP, N), f32),
            pltpu.VMEM((HPP, N), f32),
            pltpu.VMEM((NBUF, CH), i32),
            pltpu.VMEM((NBUF, CH), i32),
            pltpu.VMEM((NBUF, HPP, CH), i32),
            pltpu.VMEM((NBUF, HPP, CH), i32),
            pltpu.VMEM((NBUF, HPP, CH, FOUT), f32),
            pltpu.VMEM((NBUF, HPP, CH), f32),
            pltpu.VMEM((DEN_PAD, FOUT), f32),
            pltpu.VMEM((DEN_PAD // CH, CH), i32),
            pltpu.VMEM((ROWS_PER_TILE, FOUT), f32),
            pltpu.VMEM((DEN_PAD // NS, FOUT), f32),
            pltpu.VMEM_SHARED((ACC_PAD, FOUT), f32),
            pltpu.VMEM_SHARED((DEN_PAD, FOUT), f32),
            pltpu.SemaphoreType.DMA((NBUF,)),
            pltpu.SemaphoreType.DMA((NBUF,)),
            pltpu.SemaphoreType.DMA((NBUF,)),
            pltpu.SemaphoreType.DMA,
        ],
        compiler_params=cp,
    )
    return kern(es, et, scores_src, scores_tgt, feats_rows)


def kernel(x, edge_index, W, source_scorer, target_scorer):
    x = x.astype(jnp.float32)
    W = W.astype(jnp.float32)

    row_src = jnp.tile(source_scorer.reshape(H, FOUT), (1, H))
    row_tgt = jnp.tile(target_scorer.reshape(H, FOUT), (1, H))
    a_src = jnp.repeat(row_src, N // H, axis=0)
    a_tgt = jnp.repeat(row_tgt, N // H, axis=0)
    ones_bd = (jnp.arange(H * FOUT)[:, None] // FOUT ==
               jnp.arange(H)[None, :]).astype(jnp.float32)

    proj, ssrc, stgt = _proj_scores(x, W, a_src, a_tgt, ones_bd)

    feats_rows = proj.reshape(H * N, FOUT)
    scores_src = ssrc.reshape(H // HPP, HPP, N)
    scores_tgt = stgt.reshape(H // HPP, HPP, N)

    ei = edge_index.astype(jnp.int32)
    es = ei[:, 0]
    et = ei[:, 1]

    out = _edge_pass(es, et, scores_src, scores_tgt, feats_rows)
    return out.reshape(H, N, FOUT)
